# baseline (speedup 1.0000x reference)
import numpy as np
import jax
import jax.numpy as jnp
from jax.experimental import pallas as pl
from jax.experimental.pallas import tpu as pltpu

_LOG_SQRT_2PI = np.float32(0.5 * np.log(2.0 * np.pi))


def _diag_kernel_r3(z1_ref, z2_ref, out_ref, acc_ref, *, nsteps, denom):
    K = z2_ref.shape[0]
    M = z2_ref.shape[2]
    i = pl.program_id(0)
    z1b = z1_ref[...]
    means = z1b[:, :M]
    logvar = z1b[:, M:]
    inv_std = jnp.exp(-0.5 * logvar)
    base = -0.5 * logvar - _LOG_SQRT_2PI
    part = jnp.zeros_like(base)
    for k in range(K):
        d = (z2_ref[k] - means) * inv_std
        part = part + jnp.exp(base - 0.5 * d * d)
    RB = part.shape[0]
    vec = part.reshape(RB // 8, 8, M // 128, 128).sum(axis=(0, 2))

    @pl.when(i == 0)
    def _init():
        acc_ref[...] = vec

    @pl.when(i > 0)
    def _accum():
        acc_ref[...] = acc_ref[...] + vec

    @pl.when(i == nsteps - 1)
    def _final():
        out_ref[...] = (jnp.sum(acc_ref[...]) / denom).reshape(1, 1)


def kernel(z1, z2):
    B = z1.shape[0]
    M = z2.shape[1]
    K = z2.shape[0] // B
    z2r = z2.reshape(K, B, M)
    NSTEPS = 4
    RB = B // NSTEPS
    import functools
    out = pl.pallas_call(
        functools.partial(_diag_kernel_r3, nsteps=NSTEPS,
                          denom=np.float32(B * K * M)),
        grid=(NSTEPS,),
        in_specs=[
            pl.BlockSpec((RB, 2 * M), lambda i: (i, 0)),
            pl.BlockSpec((K, RB, M), lambda i: (0, i, 0)),
        ],
        out_specs=pl.BlockSpec((1, 1), lambda i: (0, 0)),
        out_shape=jax.ShapeDtypeStruct((1, 1), jnp.float32),
        scratch_shapes=[pltpu.VMEM((8, 128), jnp.float32)],
        compiler_params=pltpu.CompilerParams(
            dimension_semantics=("arbitrary",),
        ),
    )(z1, z2r)
    return out.reshape(())


# speedup vs baseline: 1.1777x; 1.1777x over previous
import numpy as np
import jax
import jax.numpy as jnp
from jax.experimental import pallas as pl
from jax.experimental.pallas import tpu as pltpu

_LOG_SQRT_2PI = np.float32(0.5 * np.log(2.0 * np.pi))


def _diag_kernel_r3(z1_ref, z2_ref, out_ref, acc_ref, *, nsteps, denom):
    K = z2_ref.shape[0]
    M = z2_ref.shape[2]
    i = pl.program_id(0)
    z1b = z1_ref[...]
    means = z1b[:, :M]
    logvar = z1b[:, M:]
    inv_std = jnp.exp(-0.5 * logvar)
    base = -0.5 * logvar - _LOG_SQRT_2PI
    part = jnp.zeros_like(base)
    for k in range(K):
        d = (z2_ref[k] - means) * inv_std
        part = part + jnp.exp(base - 0.5 * d * d)
    RB = part.shape[0]
    vec = part.reshape(RB // 8, 8, M // 128, 128).sum(axis=(0, 2))

    @pl.when(i == 0)
    def _init():
        acc_ref[...] = vec

    @pl.when(i > 0)
    def _accum():
        acc_ref[...] = acc_ref[...] + vec

    @pl.when(i == nsteps - 1)
    def _final():
        out_ref[...] = (jnp.sum(acc_ref[...]) / denom).reshape(1, 1)


def kernel(z1, z2):
    B = z1.shape[0]
    M = z2.shape[1]
    K = z2.shape[0] // B
    z2r = z2.reshape(K, B, M)
    NSTEPS = 2
    RB = B // NSTEPS
    import functools
    out = pl.pallas_call(
        functools.partial(_diag_kernel_r3, nsteps=NSTEPS,
                          denom=np.float32(B * K * M)),
        grid=(NSTEPS,),
        in_specs=[
            pl.BlockSpec((RB, 2 * M), lambda i: (i, 0)),
            pl.BlockSpec((K, RB, M), lambda i: (0, i, 0)),
        ],
        out_specs=pl.BlockSpec((1, 1), lambda i: (0, 0)),
        out_shape=jax.ShapeDtypeStruct((1, 1), jnp.float32),
        scratch_shapes=[pltpu.VMEM((8, 128), jnp.float32)],
        compiler_params=pltpu.CompilerParams(
            dimension_semantics=("arbitrary",),
        ),
    )(z1, z2r)
    return out.reshape(())


# speedup vs baseline: 1.1923x; 1.0124x over previous
import numpy as np
import jax
import jax.numpy as jnp
from jax.experimental import pallas as pl
from jax.experimental.pallas import tpu as pltpu

_LOG_SQRT_2PI = np.float32(0.5 * np.log(2.0 * np.pi))


def _diag_kernel_r5(z1_hbm, z2_hbm, out_ref,
                    z1a, z1b, k0a, k0b, k1a, k1b, sems, *, denom):
    B2, M = z2_hbm.shape
    B = B2 // 2
    H = B // 2

    c_z1a = pltpu.make_async_copy(z1_hbm.at[pl.ds(0, H)], z1a, sems.at[0])
    c_z1b = pltpu.make_async_copy(z1_hbm.at[pl.ds(H, H)], z1b, sems.at[1])
    c_k0a = pltpu.make_async_copy(z2_hbm.at[pl.ds(0, H)], k0a, sems.at[2])
    c_k0b = pltpu.make_async_copy(z2_hbm.at[pl.ds(H, H)], k0b, sems.at[3])
    c_k1a = pltpu.make_async_copy(z2_hbm.at[pl.ds(B, H)], k1a, sems.at[4])
    c_k1b = pltpu.make_async_copy(z2_hbm.at[pl.ds(B + H, H)], k1b, sems.at[5])
    for c in (c_z1a, c_z1b, c_k0a, c_k0b, c_k1a, c_k1b):
        c.start()

    def half(z1_buf, ka_copy, ka_buf, kb_copy, kb_buf):
        z1h = z1_buf[...]
        means = z1h[:, :M]
        logvar = z1h[:, M:]
        inv_std = jnp.exp(-0.5 * logvar)
        base = -0.5 * logvar - _LOG_SQRT_2PI
        ka_copy.wait()
        da = (ka_buf[...] - means) * inv_std
        part = jnp.exp(base - 0.5 * da * da)
        kb_copy.wait()
        db = (kb_buf[...] - means) * inv_std
        part = part + jnp.exp(base - 0.5 * db * db)
        return part.reshape(H // 8, 8, M // 128, 128).sum(axis=(0, 2))

    c_z1a.wait()
    vec = half(z1a, c_k0a, k0a, c_k1a, k1a)
    c_z1b.wait()
    vec = vec + half(z1b, c_k0b, k0b, c_k1b, k1b)
    out_ref[...] = (jnp.sum(vec) / denom).reshape(1, 1)


def kernel(z1, z2):
    B = z1.shape[0]
    M = z2.shape[1]
    K = z2.shape[0] // B
    H = B // 2
    import functools
    out = pl.pallas_call(
        functools.partial(_diag_kernel_r5, denom=np.float32(B * K * M)),
        in_specs=[
            pl.BlockSpec(memory_space=pl.ANY),
            pl.BlockSpec(memory_space=pl.ANY),
        ],
        out_specs=pl.BlockSpec((1, 1), lambda: (0, 0)),
        out_shape=jax.ShapeDtypeStruct((1, 1), jnp.float32),
        scratch_shapes=[
            pltpu.VMEM((H, 2 * M), jnp.float32),
            pltpu.VMEM((H, 2 * M), jnp.float32),
            pltpu.VMEM((H, M), jnp.float32),
            pltpu.VMEM((H, M), jnp.float32),
            pltpu.VMEM((H, M), jnp.float32),
            pltpu.VMEM((H, M), jnp.float32),
            pltpu.SemaphoreType.DMA((6,)),
        ],
    )(z1, z2)
    return out.reshape(())


# speedup vs baseline: 1.1935x; 1.0010x over previous
import numpy as np
import jax
import jax.numpy as jnp
from jax.experimental import pallas as pl
from jax.experimental.pallas import tpu as pltpu

_LOG_SQRT_2PI = np.float32(0.5 * np.log(2.0 * np.pi))
_LOG2E = np.float32(np.log2(np.e))


def _diag_kernel_r7(z1_ref, z2_ref, out_ref, *, denom):
    K = z2_ref.shape[0]
    M = z2_ref.shape[2]
    z1b = z1_ref[...]
    means = z1b[:, :M]
    logvar = z1b[:, M:]
    inv_var = jnp.exp(-logvar)
    base = -0.5 * logvar - _LOG_SQRT_2PI
    a = (-0.5 * _LOG2E) * inv_var
    b = means * (_LOG2E * inv_var)
    c = _LOG2E * base - (0.5 * means) * b
    part = jnp.zeros_like(base)
    for k in range(K):
        x = z2_ref[k]
        part = part + jnp.exp2((a * x + b) * x + c)
    vec = part
    acc = jnp.sum(vec)
    out_ref[...] = (acc / denom).reshape(1, 1)


def kernel(z1, z2):
    B = z1.shape[0]
    M = z2.shape[1]
    K = z2.shape[0] // B
    z2r = z2.reshape(K, B, M)
    import functools
    out = pl.pallas_call(
        functools.partial(_diag_kernel_r7, denom=np.float32(B * K * M)),
        out_shape=jax.ShapeDtypeStruct((1, 1), jnp.float32),
    )(z1, z2r)
    return out.reshape(())


# speedup vs baseline: 1.3440x; 1.1262x over previous
import numpy as np
import jax
import jax.numpy as jnp
from jax.experimental import pallas as pl

_LOG_SQRT_2PI = np.float32(0.5 * np.log(2.0 * np.pi))


def _diag_likelihood_kernel(z1_ref, z2_ref, out_ref):
    K, B, M = z2_ref.shape
    z1b = z1_ref[...]
    means = z1b[:, :M]
    logvar = z1b[:, M:]
    inv_std = jnp.exp(-0.5 * logvar)
    base = -0.5 * logvar - _LOG_SQRT_2PI
    acc = jnp.float32(0.0)
    for k in range(K):
        d = (z2_ref[k] - means) * inv_std
        acc = acc + jnp.sum(jnp.exp(base - 0.5 * d * d))
    out_ref[...] = (acc / np.float32(B * K * M)).reshape(1, 1)


def kernel(z1, z2):
    B = z1.shape[0]
    M = z2.shape[1]
    K = z2.shape[0] // B
    z2r = z2.reshape(K, B, M)
    out = pl.pallas_call(
        _diag_likelihood_kernel,
        out_shape=jax.ShapeDtypeStruct((1, 1), jnp.float32),
    )(z1, z2r)
    return out.reshape(())
